# baseline (speedup 1.0000x reference)
import functools

import jax
import jax.numpy as jnp
from jax import lax
from jax.experimental import pallas as pl
from jax.experimental.pallas import tpu as pltpu
from jax.experimental.pallas import tpu_sc as plsc

_NC = 2
_NS = 16
_L = 16


@functools.lru_cache(maxsize=None)
def _build_pe_kernel(B, C, H, W):
    R = 2 * C
    K = H * W
    NW = _NC * _NS
    assert R % NW == 0
    ROWS_W = R // NW
    CHUNK = ROWS_W * K
    assert C % (NW // 2) == 0 and ROWS_W <= C
    assert W % _L == 0 and K % _L == 0

    mesh = plsc.VectorSubcoreMesh(core_axis_name="c", subcore_axis_name="s")

    @functools.partial(
        pl.kernel,
        out_type=jax.ShapeDtypeStruct((B, R * K), jnp.float32),
        mesh=mesh,
        scratch_types=[
            pltpu.VMEM((H, C), jnp.float32),
            pltpu.VMEM((CHUNK,), jnp.float32),
            pltpu.SemaphoreType.DMA,
        ],
        compiler_params=pltpu.CompilerParams(
            use_tc_tiling_on_sc=False, needs_layout_passes=False),
    )
    def pe_kernel(emb_h_hbm, emb_w_hbm, out_hbm, emb_v, chunk_v, sem):
        cid = lax.axis_index("c")
        sid = lax.axis_index("s")
        wid = cid * _NS + sid
        is_top = wid < (NW // 2)
        c0 = jnp.where(is_top, wid, wid - (NW // 2)) * ROWS_W

        lane = lax.iota(jnp.int32, _L)

        @pl.when(is_top)
        def _():
            pltpu.sync_copy(emb_h_hbm.at[pl.ds(0, W)], emb_v)
            for t in range(ROWS_W):
                c_idx = jnp.full((_L,), c0 + t, jnp.int32)
                for rep in range(W // _L):
                    vec = plsc.load_gather(emb_v, [lane + rep * _L, c_idx])
                    for m in range(K // W):
                        chunk_v[pl.ds(t * K + m * W + rep * _L, _L)] = vec

        @pl.when(jnp.logical_not(is_top))
        def _():
            pltpu.sync_copy(emb_w_hbm.at[pl.ds(0, H)], emb_v)
            for t in range(ROWS_W):
                c_idx = jnp.full((_L,), c0 + t, jnp.int32)
                for i in range(H):
                    vec = plsc.load_gather(
                        emb_v, [jnp.full((_L,), i, jnp.int32), c_idx])
                    for rep in range(W // _L):
                        chunk_v[pl.ds(t * K + i * W + rep * _L, _L)] = vec

        base = wid * CHUNK
        copies = [
            pltpu.make_async_copy(
                chunk_v, out_hbm.at[b, pl.ds(base, CHUNK)], sem)
            for b in range(B)
        ]
        for cp in copies:
            cp.start()
        for cp in copies:
            cp.wait()

    return pe_kernel


def kernel(x, emb_h, emb_w):
    B = x.shape[0]
    H, W = x.shape[-2], x.shape[-1]
    C = emb_h.shape[1]
    out = _build_pe_kernel(B, C, H, W)(emb_h, emb_w)
    return out.reshape(B, 2 * C, H, W)


# speedup vs baseline: 5.7475x; 5.7475x over previous
import functools

import jax
import jax.numpy as jnp
from jax import lax
from jax.experimental import pallas as pl
from jax.experimental.pallas import tpu as pltpu
from jax.experimental.pallas import tpu_sc as plsc

_NC = 2
_NS = 16
_L = 16


@functools.lru_cache(maxsize=None)
def _build_pe_kernel(B, C, H, W):
    NW = _NC * _NS
    assert H == NW
    assert W % 8 == 0 and (2 * C) % 128 == 0 and C % 128 == 0
    JT = W // 8
    CT = (2 * C) // 128
    BLK_ROWS = JT * CT * 8
    N = B * H * BLK_ROWS

    mesh = plsc.VectorSubcoreMesh(core_axis_name="c", subcore_axis_name="s")

    @functools.partial(
        pl.kernel,
        out_type=jax.ShapeDtypeStruct((N, 128), jnp.float32),
        mesh=mesh,
        scratch_types=[
            pltpu.VMEM((BLK_ROWS, 128), jnp.float32),
            pltpu.VMEM((1, C), jnp.float32),
            pltpu.SemaphoreType.DMA,
        ],
        compiler_params=pltpu.CompilerParams(
            use_tc_tiling_on_sc=False, needs_layout_passes=False),
    )
    def pe_kernel(emb_h_hbm, emb_w_hbm, out_hbm, blk_v, ew_v, sem):
        cid = lax.axis_index("c")
        sid = lax.axis_index("s")
        wid = cid * _NS + sid

        pltpu.sync_copy(emb_w_hbm.at[pl.ds(wid, 1)], ew_v)

        for jt in range(JT):
            for ct in range(CT // 2):
                pltpu.sync_copy(
                    emb_h_hbm.at[pl.ds(jt * 8, 8), pl.ds(ct * 128, 128)],
                    blk_v.at[pl.ds(jt * CT * 8 + ct * 8, 8)])

        for ct in range(CT // 2):
            vecs = [ew_v[0, pl.ds(ct * 128 + v * _L, _L)]
                    for v in range(128 // _L)]
            for jt in range(JT):
                base = jt * CT * 8 + (CT // 2 + ct) * 8
                for jr in range(8):
                    for v in range(128 // _L):
                        blk_v[base + jr, pl.ds(v * _L, _L)] = vecs[v]

        copies = [
            pltpu.make_async_copy(
                blk_v,
                out_hbm.at[pl.ds((b * H + wid) * BLK_ROWS, BLK_ROWS)],
                sem)
            for b in range(B)
        ]
        for cp in copies:
            cp.start()
        for cp in copies:
            cp.wait()

    return pe_kernel


def kernel(x, emb_h, emb_w):
    B = x.shape[0]
    H, W = x.shape[-2], x.shape[-1]
    C = emb_h.shape[1]
    out2 = _build_pe_kernel(B, C, H, W)(emb_h, emb_w)
    out6 = out2.reshape(B, H, W // 8, (2 * C) // 128, 8, 128)
    return out6.transpose(0, 3, 5, 1, 2, 4).reshape(B, 2 * C, H, W)


# speedup vs baseline: 6.2005x; 1.0788x over previous
import functools

import jax
import jax.numpy as jnp
from jax import lax
from jax.experimental import pallas as pl
from jax.experimental.pallas import tpu as pltpu
from jax.experimental.pallas import tpu_sc as plsc

_NC = 2
_NS = 16
_L = 16


def _geom(C, H, W):
    JT = W // 8
    CT = (2 * C) // 128
    return JT, CT, JT * CT * 8


@functools.lru_cache(maxsize=None)
def _build_sc_kernel(B, C, H, W):
    NW = _NC * _NS
    NP = NW // 2
    assert H == NW and W % 8 == 0 and C % 128 == 0 and B % 2 == 0
    JT, CT, BLK_ROWS = _geom(C, H, W)
    N = B * H * BLK_ROWS
    B2 = B // 2

    mesh = plsc.VectorSubcoreMesh(core_axis_name="c", subcore_axis_name="s")

    @functools.partial(
        pl.kernel,
        out_type=jax.ShapeDtypeStruct((N, 128), jnp.float32),
        mesh=mesh,
        scratch_types=[
            pltpu.VMEM((2 * BLK_ROWS, 128), jnp.float32),
            pltpu.VMEM((2, C), jnp.float32),
            pltpu.SemaphoreType.DMA,
        ],
        compiler_params=pltpu.CompilerParams(
            use_tc_tiling_on_sc=False, needs_layout_passes=False),
    )
    def sc_kernel(emb_h_hbm, emb_w_hbm, out_hbm, blk_v, ew_v, sem):
        cid = lax.axis_index("c")
        sid = lax.axis_index("s")
        wid = cid * _NS + sid
        pair = wid % NP
        b_lo = (wid // NP) * B2

        stage = [pltpu.make_async_copy(
            emb_w_hbm.at[pl.ds(pair * 2, 2)], ew_v, sem)]
        for h2 in range(2):
            for jt in range(JT):
                for ct in range(CT // 2):
                    stage.append(pltpu.make_async_copy(
                        emb_h_hbm.at[pl.ds(jt * 8, 8), pl.ds(ct * 128, 128)],
                        blk_v.at[pl.ds(h2 * BLK_ROWS + jt * CT * 8 + ct * 8,
                                       8)],
                        sem))
        for cp in stage:
            cp.start()
        for cp in stage:
            cp.wait()

        for h2 in range(2):
            for ct in range(CT // 2):
                vecs = [ew_v[h2, pl.ds(ct * 128 + v * _L, _L)]
                        for v in range(128 // _L)]
                for jt in range(JT):
                    base = (h2 * BLK_ROWS + jt * CT * 8
                            + (CT // 2 + ct) * 8)
                    for jr in range(8):
                        for v in range(128 // _L):
                            blk_v[base + jr, pl.ds(v * _L, _L)] = vecs[v]

        copies = [
            pltpu.make_async_copy(
                blk_v,
                out_hbm.at[pl.ds(((b_lo + k) * H + pair * 2) * BLK_ROWS,
                                 2 * BLK_ROWS)],
                sem)
            for k in range(B2)
        ]
        for cp in copies:
            cp.start()
        for cp in copies:
            cp.wait()

    return sc_kernel


def kernel(x, emb_h, emb_w):
    B = x.shape[0]
    H, W = x.shape[-2], x.shape[-1]
    C = emb_h.shape[1]
    out2 = _build_sc_kernel(B, C, H, W)(emb_h, emb_w)
    out6 = out2.reshape(B, H, W // 8, (2 * C) // 128, 8, 128)
    return out6.transpose(0, 3, 5, 1, 2, 4).reshape(B, 2 * C, H, W)


# speedup vs baseline: 6.4831x; 1.0456x over previous
import functools

import jax
import jax.numpy as jnp
from jax import lax
from jax.experimental import pallas as pl
from jax.experimental.pallas import tpu as pltpu
from jax.experimental.pallas import tpu_sc as plsc

_NC = 2
_NS = 16
_L = 16

_SC_BATCHES = 64


def _geom(C, H, W):
    JT = W // 8
    CT = (2 * C) // 128
    return JT, CT, JT * CT * 8


@functools.lru_cache(maxsize=None)
def _build_sc_kernel(B, C, H, W):
    NW = _NC * _NS
    assert H == NW and W % 8 == 0 and C % 128 == 0
    JT, CT, BLK_ROWS = _geom(C, H, W)
    N = B * H * BLK_ROWS

    mesh = plsc.VectorSubcoreMesh(core_axis_name="c", subcore_axis_name="s")

    @functools.partial(
        pl.kernel,
        out_type=jax.ShapeDtypeStruct((N, 128), jnp.float32),
        mesh=mesh,
        scratch_types=[
            pltpu.VMEM((BLK_ROWS, 128), jnp.float32),
            pltpu.VMEM((1, C), jnp.float32),
            pltpu.SemaphoreType.DMA,
        ],
        compiler_params=pltpu.CompilerParams(
            use_tc_tiling_on_sc=False, needs_layout_passes=False,
            disable_bounds_checks=True, disable_semaphore_checks=True),
    )
    def sc_kernel(emb_h_hbm, emb_w_hbm, out_hbm, blk_v, ew_v, sem):
        cid = lax.axis_index("c")
        sid = lax.axis_index("s")
        wid = cid * _NS + sid

        stage = [pltpu.make_async_copy(emb_w_hbm.at[pl.ds(wid, 1)], ew_v, sem)]
        for jt in range(JT):
            for ct in range(CT // 2):
                stage.append(pltpu.make_async_copy(
                    emb_h_hbm.at[pl.ds(jt * 8, 8), pl.ds(ct * 128, 128)],
                    blk_v.at[pl.ds(jt * CT * 8 + ct * 8, 8)], sem))
        for cp in stage:
            cp.start()
        for cp in stage:
            cp.wait()

        for ct in range(CT // 2):
            vecs = [ew_v[0, pl.ds(ct * 128 + v * _L, _L)]
                    for v in range(128 // _L)]
            for jt in range(JT):
                base = jt * CT * 8 + (CT // 2 + ct) * 8
                for jr in range(8):
                    for v in range(128 // _L):
                        blk_v[base + jr, pl.ds(v * _L, _L)] = vecs[v]

        copies = [
            pltpu.make_async_copy(
                blk_v,
                out_hbm.at[pl.ds((b * H + wid) * BLK_ROWS, BLK_ROWS)],
                sem)
            for b in range(B)
        ]
        for cp in copies:
            cp.start()
        for cp in copies:
            cp.wait()

    return sc_kernel


@functools.lru_cache(maxsize=None)
def _build_tc_kernel(B, C, H, W):
    JT, CT, BLK_ROWS = _geom(C, H, W)
    PLANE = H * BLK_ROWS

    def tc_body(eh_ref, ew_ref, out_ref, plane_ref):
        @pl.when(pl.program_id(0) == 0)
        def _():
            eh = eh_ref[0:W, :]
            ew = ew_ref[0:H, :]
            top = jnp.stack(
                [eh[:, ct * 128:(ct + 1) * 128].reshape(JT, 8, 128)
                 for ct in range(CT // 2)], axis=1)
            bot = jnp.stack(
                [ew[:, ct * 128:(ct + 1) * 128]
                 for ct in range(CT // 2)], axis=1)
            top_b = jnp.broadcast_to(
                top[None], (H, JT, CT // 2, 8, 128))
            bot_b = jnp.broadcast_to(
                bot[:, None, :, None, :], (H, JT, CT // 2, 8, 128))
            blk = jnp.concatenate([top_b, bot_b], axis=2)
            plane_ref[...] = blk.reshape(PLANE, 128)

        out_ref[...] = plane_ref[...]

    return pl.pallas_call(
        tc_body,
        grid=(B,),
        in_specs=[
            pl.BlockSpec((50, C), lambda b: (0, 0)),
            pl.BlockSpec((50, C), lambda b: (0, 0)),
        ],
        out_specs=pl.BlockSpec((PLANE, 128), lambda b: (b, 0)),
        out_shape=jax.ShapeDtypeStruct((B * PLANE, 128), jnp.float32),
        scratch_shapes=[pltpu.VMEM((PLANE, 128), jnp.float32)],
    )


def kernel(x, emb_h, emb_w):
    B = x.shape[0]
    H, W = x.shape[-2], x.shape[-1]
    C = emb_h.shape[1]
    b_sc = min(_SC_BATCHES, B) if H == _NC * _NS else 0
    parts = []
    if b_sc:
        parts.append(_build_sc_kernel(b_sc, C, H, W)(emb_h, emb_w))
    if B - b_sc:
        parts.append(_build_tc_kernel(B - b_sc, C, H, W)(emb_h, emb_w))
    out2 = parts[0] if len(parts) == 1 else jnp.concatenate(parts, axis=0)
    out6 = out2.reshape(B, H, W // 8, (2 * C) // 128, 8, 128)
    return out6.transpose(0, 3, 5, 1, 2, 4).reshape(B, 2 * C, H, W)


# speedup vs baseline: 6.5897x; 1.0165x over previous
import functools

import jax
import jax.numpy as jnp
from jax import lax
from jax.experimental import pallas as pl
from jax.experimental.pallas import tpu as pltpu
from jax.experimental.pallas import tpu_sc as plsc

_NC = 2
_NS = 16
_L = 16


def _geom(C, H, W):
    JT = W // 8
    CT = (2 * C) // 128
    return JT, CT, JT * CT * 8


@functools.lru_cache(maxsize=None)
def _build_sc_kernel(B, C, H, W):
    NW = _NC * _NS
    assert H == NW and W % 8 == 0 and C % 128 == 0
    JT, CT, BLK_ROWS = _geom(C, H, W)
    N = B * H * BLK_ROWS

    mesh = plsc.VectorSubcoreMesh(core_axis_name="c", subcore_axis_name="s")

    @functools.partial(
        pl.kernel,
        out_type=jax.ShapeDtypeStruct((N, 128), jnp.float32),
        mesh=mesh,
        scratch_types=[
            pltpu.VMEM((BLK_ROWS, 128), jnp.float32),
            pltpu.VMEM((1, C), jnp.float32),
            pltpu.SemaphoreType.DMA,
        ],
        compiler_params=pltpu.CompilerParams(
            use_tc_tiling_on_sc=False, needs_layout_passes=False),
    )
    def sc_kernel(emb_h_hbm, emb_w_hbm, out_hbm, blk_v, ew_v, sem):
        cid = lax.axis_index("c")
        sid = lax.axis_index("s")
        wid = cid * _NS + sid

        stage = [pltpu.make_async_copy(emb_w_hbm.at[pl.ds(wid, 1)], ew_v, sem)]
        for jt in range(JT):
            for ct in range(CT // 2):
                stage.append(pltpu.make_async_copy(
                    emb_h_hbm.at[pl.ds(jt * 8, 8), pl.ds(ct * 128, 128)],
                    blk_v.at[pl.ds(jt * CT * 8 + ct * 8, 8)], sem))
        for cp in stage:
            cp.start()
        for cp in stage:
            cp.wait()

        for ct in range(CT // 2):
            vecs = [ew_v[0, pl.ds(ct * 128 + v * _L, _L)]
                    for v in range(128 // _L)]
            for jt in range(JT):
                base = jt * CT * 8 + (CT // 2 + ct) * 8
                for jr in range(8):
                    for v in range(128 // _L):
                        blk_v[base + jr, pl.ds(v * _L, _L)] = vecs[v]

        copies = [
            pltpu.make_async_copy(
                blk_v,
                out_hbm.at[pl.ds((b * H + wid) * BLK_ROWS, BLK_ROWS)],
                sem)
            for b in range(B)
        ]
        for cp in copies:
            cp.start()
        for cp in copies:
            cp.wait()

    return sc_kernel


def kernel(x, emb_h, emb_w):
    B = x.shape[0]
    H, W = x.shape[-2], x.shape[-1]
    C = emb_h.shape[1]
    out2 = _build_sc_kernel(B, C, H, W)(emb_h, emb_w)
    out6 = out2.reshape(B, H, W // 8, (2 * C) // 128, 8, 128)
    return out6.transpose(0, 3, 5, 1, 2, 4).reshape(B, 2 * C, H, W)
